# baseline (speedup 1.0000x reference)
import functools

import jax
import jax.numpy as jnp
from jax import lax
from jax.experimental import pallas as pl
from jax.experimental.pallas import tpu as pltpu
from jax.experimental.pallas import tpu_sc as plsc

_NUM_TABLES = 4
_EMB = 32
_NW = 32
_CHUNK = 128
_LANES = 16


@functools.partial(jax.jit, static_argnames=("batch",))
def _gather_concat(idx2d, t0, t1, t2, t3, *, batch):
    b_per_w = batch // _NW
    n_chunks = b_per_w // _CHUNK

    mesh = plsc.VectorSubcoreMesh(core_axis_name="c", subcore_axis_name="s")

    @functools.partial(
        pl.kernel,
        out_type=jax.ShapeDtypeStruct((batch, _NUM_TABLES * _EMB),
                                      jnp.float32),
        mesh=mesh,
        scratch_types=[
            pltpu.VMEM((n_chunks, _CHUNK), jnp.int32),
            pltpu.VMEM((n_chunks, _CHUNK), jnp.int32),
            pltpu.VMEM((_CHUNK, 4 * _EMB), jnp.float32),
            pltpu.VMEM((_CHUNK, 4 * _EMB), jnp.float32),
            pltpu.VMEM((_CHUNK, 4 * _EMB), jnp.float32),
            pltpu.VMEM((_CHUNK, 4 * _EMB), jnp.float32),
            pltpu.VMEM((_CHUNK, _NUM_TABLES * _EMB), jnp.float32),
            pltpu.SemaphoreType.DMA,
        ],
    )
    def k(idx_hbm, t0_hbm, t1_hbm, t2_hbm, t3_hbm, out_hbm,
          idx_v, qidx_v, s0, s1, s2, s3, blk, sem):
        wid = lax.axis_index("s") * 2 + lax.axis_index("c")
        base = wid * b_per_w
        pltpu.sync_copy(idx_hbm.at[pl.ds(wid * n_chunks, n_chunks)], idx_v)
        for j in range(n_chunks):
            for c in range(_CHUNK // _LANES):
                sl = pl.ds(c * _LANES, _LANES)
                v = idx_v[j, sl]
                qidx_v[j, sl] = (
                    jax.lax.shift_left(
                        jax.lax.shift_right_logical(v, 5), 3)
                    | (v & 7))
        for j in range(n_chunks):
            copies = [
                pltpu.async_copy(t.at[qidx_v.at[j]], s, sem)
                for t, s in ((t0_hbm, s0), (t1_hbm, s1),
                             (t2_hbm, s2), (t3_hbm, s3))
            ]
            for cp in copies:
                cp.wait()
            for m, s in enumerate((s0, s1, s2, s3)):
                def body(g, _, s=s, m=m, j=j):
                    ivec = idx_v[j, pl.ds(g * _LANES, _LANES)]
                    offv = (jax.lax.shift_right_logical(ivec, 3) & 3) * _EMB
                    for l in range(_LANES):
                        kk = g * _LANES + l
                        off = offv[l]
                        blk[kk, pl.ds(m * _EMB, _LANES)] = \
                            s[kk, pl.ds(off, _LANES)]
                        blk[kk, pl.ds(m * _EMB + _LANES, _LANES)] = \
                            s[kk, pl.ds(off + _LANES, _LANES)]
                    return 0
                lax.fori_loop(0, _CHUNK // _LANES, body, 0)
            pltpu.sync_copy(
                blk, out_hbm.at[pl.ds(base + j * _CHUNK, _CHUNK)])

    return k(idx2d, t0, t1, t2, t3)


def _packed_view(t):
    v = t.shape[0]
    return (t.reshape(v // 32, 4, 8, _EMB)
            .transpose(0, 2, 1, 3)
            .reshape(v // 4, 4 * _EMB))


def kernel(indexes, table0, table1, table2, table3):
    batch = indexes.shape[0]
    idx2d = indexes.astype(jnp.int32).reshape(batch // _CHUNK, _CHUNK)
    tables = [_packed_view(t) for t in (table0, table1, table2, table3)]
    return _gather_concat(idx2d, *tables, batch=batch)


# speedup vs baseline: 2.1548x; 2.1548x over previous
import functools

import jax
import jax.numpy as jnp
from jax import lax
from jax.experimental import pallas as pl
from jax.experimental.pallas import tpu as pltpu
from jax.experimental.pallas import tpu_sc as plsc
from jax.experimental import layout as jax_layout

_NUM_TABLES = 4
_EMB = 32
_NW = 32
_CHUNK = 128
_LANES = 16


@functools.partial(jax.jit, static_argnames=("batch",))
def _gather_concat(idx_flat, t0, t1, t2, t3, *, batch):
    b_per_w = batch // _NW
    n_chunks = b_per_w // _CHUNK

    mesh = plsc.VectorSubcoreMesh(core_axis_name="c", subcore_axis_name="s")

    @functools.partial(
        pl.kernel,
        out_type=jax.ShapeDtypeStruct((batch * _NUM_TABLES, _EMB),
                                      jnp.float32),
        mesh=mesh,
        scratch_types=[
            pltpu.VMEM((n_chunks, _CHUNK), jnp.int32),
            pltpu.VMEM((_NUM_TABLES * n_chunks, _CHUNK), jnp.int32),
            pltpu.VMEM((b_per_w, _EMB), jnp.float32),
            pltpu.VMEM((b_per_w, _EMB), jnp.float32),
            pltpu.VMEM((b_per_w, _EMB), jnp.float32),
            pltpu.VMEM((b_per_w, _EMB), jnp.float32),
            pltpu.SemaphoreType.DMA,
        ],
        compiler_params=pltpu.CompilerParams(use_tc_tiling_on_sc=False),
    )
    def k(idx_hbm, t0_hbm, t1_hbm, t2_hbm, t3_hbm, out_hbm,
          idx_v, sidx_v, r0, r1, r2, r3, sem):
        wid = lax.axis_index("s") * 2 + lax.axis_index("c")
        base = wid * b_per_w
        for j in range(n_chunks):
            pltpu.sync_copy(idx_hbm.at[pl.ds(base + j * _CHUNK, _CHUNK)],
                            idx_v.at[j])
        gathers = []
        for t, r in ((t0_hbm, r0), (t1_hbm, r1), (t2_hbm, r2), (t3_hbm, r3)):
            for j in range(n_chunks):
                gathers.append(
                    pltpu.async_copy(
                        t.at[idx_v.at[j]],
                        r.at[pl.ds(j * _CHUNK, _CHUNK)],
                        sem,
                    ))
        lanes = lax.broadcasted_iota(jnp.int32, (_LANES,), 0)
        for j in range(n_chunks):
            for c in range(_CHUNK // _LANES):
                gv4 = (base + j * _CHUNK + c * _LANES) * _NUM_TABLES \
                    + lanes * _NUM_TABLES
                for m in range(_NUM_TABLES):
                    sidx_v[m * n_chunks + j, pl.ds(c * _LANES, _LANES)] = \
                        gv4 + m
        for g in gathers:
            g.wait()
        scatters = []
        for m, r in enumerate((r0, r1, r2, r3)):
            for j in range(n_chunks):
                scatters.append(
                    pltpu.async_copy(
                        r.at[pl.ds(j * _CHUNK, _CHUNK)],
                        out_hbm.at[sidx_v.at[m * n_chunks + j]],
                        sem,
                    ))
        for s in scatters:
            s.wait()

    lay = jax_layout.Layout((0, 1), ((8,),))
    t0, t1, t2, t3 = (
        jax_layout.with_layout_constraint(t, lay) for t in (t0, t1, t2, t3))
    return k(idx_flat, t0, t1, t2, t3)


def kernel(indexes, table0, table1, table2, table3):
    batch = indexes.shape[0]
    out = _gather_concat(indexes.astype(jnp.int32), table0, table1,
                         table2, table3, batch=batch)
    return out.reshape(batch, _NUM_TABLES * _EMB)
